# baseline (speedup 1.0000x reference)
import functools

import jax
import jax.numpy as jnp
from jax.experimental import pallas as pl
from jax.experimental.pallas import tpu as pltpu

HIDDEN = 2048
NUM_EXPERTS = 16
TOP_K = 2
BETA = 0.1
GAMMA = 0.1
EXPLORATION_C = 0.1

BLK = 512


def _router_body(x_ref, wt_ref, rep_ref, loads_ref, cnts_ref, tot_ref,
                 w_ref, idx_ref, aux_ref, cnt_acc, psum_acc):
    i = pl.program_id(0)
    nsteps = pl.num_programs(0)
    logits = jnp.dot(x_ref[...], wt_ref[...],
                     preferred_element_type=jnp.float32)
    tot = tot_ref[0, 0]
    bias = (BETA * rep_ref[...] - GAMMA * loads_ref[...]
            + EXPLORATION_C * jnp.sqrt(
                jnp.log(tot + 1.0) / (cnts_ref[...] + 1e-10)))
    s = logits + bias
    iota = jax.lax.broadcasted_iota(jnp.int32, (BLK, NUM_EXPERTS), 1)
    m1 = jnp.max(s, axis=1, keepdims=True)
    i1 = jnp.min(jnp.where(s == m1, iota, NUM_EXPERTS), axis=1, keepdims=True)
    s2 = jnp.where(iota == i1, -1e30, s)
    m2 = jnp.max(s2, axis=1, keepdims=True)
    i2 = jnp.min(jnp.where(s2 == m2, iota, NUM_EXPERTS), axis=1, keepdims=True)
    e2 = jnp.exp(m2 - m1)
    w1 = 1.0 / (1.0 + e2)
    w_ref[...] = jnp.concatenate([w1, 1.0 - w1], axis=1)
    idx_ref[...] = jnp.concatenate([i1, i2], axis=1)
    z = jnp.exp(s - m1)
    probs = z / jnp.sum(z, axis=1, keepdims=True)
    oh = ((iota == i1).astype(jnp.float32)
          + (iota == i2).astype(jnp.float32))

    @pl.when(i == 0)
    def _init():
        cnt_acc[...] = jnp.zeros_like(cnt_acc)
        psum_acc[...] = jnp.zeros_like(psum_acc)

    cnt_acc[...] += jnp.sum(oh, axis=0, keepdims=True)
    psum_acc[...] += jnp.sum(probs, axis=0, keepdims=True)

    @pl.when(i == nsteps - 1)
    def _fin():
        t_total = jnp.float32(BLK * nsteps)
        aux_ref[...] = (jnp.sum(cnt_acc[...] * psum_acc[...], keepdims=True)
                        * (NUM_EXPERTS / (t_total * t_total)))


@functools.partial(jax.jit, static_argnames=("interpret",))
def _run(x, W, reputation_scores, expert_loads, expert_counts,
         total_routing_decisions, interpret=False):
    B, S, H = x.shape
    T = B * S
    nsteps = T // BLK
    x2 = x.reshape(T, H)
    wt = W.T
    rep = reputation_scores.reshape(1, NUM_EXPERTS)
    loads = expert_loads.reshape(1, NUM_EXPERTS)
    cnts = expert_counts.reshape(1, NUM_EXPERTS)
    tot = total_routing_decisions.reshape(1, 1)

    w_flat, idx_flat, aux = pl.pallas_call(
        _router_body,
        grid=(nsteps,),
        in_specs=[
            pl.BlockSpec((BLK, H), lambda i: (i, 0)),
            pl.BlockSpec((H, NUM_EXPERTS), lambda i: (0, 0)),
            pl.BlockSpec((1, NUM_EXPERTS), lambda i: (0, 0)),
            pl.BlockSpec((1, NUM_EXPERTS), lambda i: (0, 0)),
            pl.BlockSpec((1, NUM_EXPERTS), lambda i: (0, 0)),
            pl.BlockSpec((1, 1), lambda i: (0, 0)),
        ],
        out_specs=[
            pl.BlockSpec((BLK, TOP_K), lambda i: (i, 0)),
            pl.BlockSpec((BLK, TOP_K), lambda i: (i, 0)),
            pl.BlockSpec((1, 1), lambda i: (0, 0)),
        ],
        out_shape=[
            jax.ShapeDtypeStruct((T, TOP_K), jnp.float32),
            jax.ShapeDtypeStruct((T, TOP_K), jnp.int32),
            jax.ShapeDtypeStruct((1, 1), jnp.float32),
        ],
        scratch_shapes=[
            pltpu.VMEM((1, NUM_EXPERTS), jnp.float32),
            pltpu.VMEM((1, NUM_EXPERTS), jnp.float32),
        ],
        interpret=interpret,
    )(x2, wt, rep, loads, cnts, tot)
    return (w_flat.reshape(B, S, TOP_K),
            idx_flat.reshape(B, S, TOP_K),
            aux[0, 0])


def kernel(x, W, reputation_scores, expert_loads, expert_counts,
           total_routing_decisions):
    return _run(x, W, reputation_scores, expert_loads, expert_counts,
                total_routing_decisions)


# speedup vs baseline: 1.1339x; 1.1339x over previous
import functools

import jax
import jax.numpy as jnp
from jax.experimental import pallas as pl
from jax.experimental.pallas import tpu as pltpu

HIDDEN = 2048
NUM_EXPERTS = 16
TOP_K = 2
BETA = 0.1
GAMMA = 0.1
EXPLORATION_C = 0.1

BLK = 1024


def _router_body(x_ref, wt_ref, rep_ref, loads_ref, cnts_ref, tot_ref,
                 w_ref, idx_ref, aux_ref, cnt_acc, psum_acc):
    i = pl.program_id(0)
    nsteps = pl.num_programs(0)
    logits = jnp.dot(x_ref[...], wt_ref[...],
                     preferred_element_type=jnp.float32)
    tot = tot_ref[0, 0]
    bias = (BETA * rep_ref[...] - GAMMA * loads_ref[...]
            + EXPLORATION_C * jnp.sqrt(
                jnp.log(tot + 1.0) / (cnts_ref[...] + 1e-10)))
    s = logits + bias
    iota = jax.lax.broadcasted_iota(jnp.int32, (BLK, NUM_EXPERTS), 1)
    m1 = jnp.max(s, axis=1, keepdims=True)
    i1 = jnp.min(jnp.where(s == m1, iota, NUM_EXPERTS), axis=1, keepdims=True)
    s2 = jnp.where(iota == i1, -1e30, s)
    m2 = jnp.max(s2, axis=1, keepdims=True)
    i2 = jnp.min(jnp.where(s2 == m2, iota, NUM_EXPERTS), axis=1, keepdims=True)
    e2 = jnp.exp(m2 - m1)
    w1 = 1.0 / (1.0 + e2)
    w_ref[...] = jnp.concatenate([w1, 1.0 - w1], axis=1)
    idx_ref[...] = jnp.concatenate([i1, i2], axis=1)
    z = jnp.exp(s - m1)
    probs = z / jnp.sum(z, axis=1, keepdims=True)
    oh = ((iota == i1).astype(jnp.float32)
          + (iota == i2).astype(jnp.float32))

    @pl.when(i == 0)
    def _init():
        cnt_acc[...] = jnp.zeros_like(cnt_acc)
        psum_acc[...] = jnp.zeros_like(psum_acc)

    cnt_acc[...] += jnp.sum(oh, axis=0, keepdims=True)
    psum_acc[...] += jnp.sum(probs, axis=0, keepdims=True)

    @pl.when(i == nsteps - 1)
    def _fin():
        t_total = jnp.float32(BLK * nsteps)
        aux_ref[...] = (jnp.sum(cnt_acc[...] * psum_acc[...], keepdims=True)
                        * (NUM_EXPERTS / (t_total * t_total)))


@functools.partial(jax.jit, static_argnames=("interpret",))
def _run(x, W, reputation_scores, expert_loads, expert_counts,
         total_routing_decisions, interpret=False):
    B, S, H = x.shape
    T = B * S
    nsteps = T // BLK
    x2 = x.reshape(T, H)
    wt = W.T
    rep = reputation_scores.reshape(1, NUM_EXPERTS)
    loads = expert_loads.reshape(1, NUM_EXPERTS)
    cnts = expert_counts.reshape(1, NUM_EXPERTS)
    tot = total_routing_decisions.reshape(1, 1)

    w_flat, idx_flat, aux = pl.pallas_call(
        _router_body,
        grid=(nsteps,),
        in_specs=[
            pl.BlockSpec((BLK, H), lambda i: (i, 0)),
            pl.BlockSpec((H, NUM_EXPERTS), lambda i: (0, 0)),
            pl.BlockSpec((1, NUM_EXPERTS), lambda i: (0, 0)),
            pl.BlockSpec((1, NUM_EXPERTS), lambda i: (0, 0)),
            pl.BlockSpec((1, NUM_EXPERTS), lambda i: (0, 0)),
            pl.BlockSpec((1, 1), lambda i: (0, 0)),
        ],
        out_specs=[
            pl.BlockSpec((BLK, TOP_K), lambda i: (i, 0)),
            pl.BlockSpec((BLK, TOP_K), lambda i: (i, 0)),
            pl.BlockSpec((1, 1), lambda i: (0, 0)),
        ],
        out_shape=[
            jax.ShapeDtypeStruct((T, TOP_K), jnp.float32),
            jax.ShapeDtypeStruct((T, TOP_K), jnp.int32),
            jax.ShapeDtypeStruct((1, 1), jnp.float32),
        ],
        scratch_shapes=[
            pltpu.VMEM((1, NUM_EXPERTS), jnp.float32),
            pltpu.VMEM((1, NUM_EXPERTS), jnp.float32),
        ],
        interpret=interpret,
    )(x2, wt, rep, loads, cnts, tot)
    return (w_flat.reshape(B, S, TOP_K),
            idx_flat.reshape(B, S, TOP_K),
            aux[0, 0])


def kernel(x, W, reputation_scores, expert_loads, expert_counts,
           total_routing_decisions):
    return _run(x, W, reputation_scores, expert_loads, expert_counts,
                total_routing_decisions)


# speedup vs baseline: 1.1573x; 1.0206x over previous
import functools

import jax
import jax.numpy as jnp
from jax.experimental import pallas as pl
from jax.experimental.pallas import tpu as pltpu

HIDDEN = 2048
NUM_EXPERTS = 16
TOP_K = 2
BETA = 0.1
GAMMA = 0.1
EXPLORATION_C = 0.1

BLK = 2048


def _router_body(x_ref, wt_ref, rep_ref, loads_ref, cnts_ref, tot_ref,
                 w_ref, idx_ref, aux_ref, cnt_acc, psum_acc):
    i = pl.program_id(0)
    nsteps = pl.num_programs(0)
    logits = jnp.dot(x_ref[...], wt_ref[...],
                     preferred_element_type=jnp.float32)
    tot = tot_ref[0, 0]
    bias = (BETA * rep_ref[...] - GAMMA * loads_ref[...]
            + EXPLORATION_C * jnp.sqrt(
                jnp.log(tot + 1.0) / (cnts_ref[...] + 1e-10)))
    s = logits + bias
    iota = jax.lax.broadcasted_iota(jnp.int32, (BLK, NUM_EXPERTS), 1)
    m1 = jnp.max(s, axis=1, keepdims=True)
    i1 = jnp.min(jnp.where(s == m1, iota, NUM_EXPERTS), axis=1, keepdims=True)
    s2 = jnp.where(iota == i1, -1e30, s)
    m2 = jnp.max(s2, axis=1, keepdims=True)
    i2 = jnp.min(jnp.where(s2 == m2, iota, NUM_EXPERTS), axis=1, keepdims=True)
    e2 = jnp.exp(m2 - m1)
    w1 = 1.0 / (1.0 + e2)
    w_ref[...] = jnp.concatenate([w1, 1.0 - w1], axis=1)
    idx_ref[...] = jnp.concatenate([i1, i2], axis=1)
    z = jnp.exp(s - m1)
    probs = z / jnp.sum(z, axis=1, keepdims=True)
    oh = ((iota == i1).astype(jnp.float32)
          + (iota == i2).astype(jnp.float32))

    @pl.when(i == 0)
    def _init():
        cnt_acc[...] = jnp.zeros_like(cnt_acc)
        psum_acc[...] = jnp.zeros_like(psum_acc)

    cnt_acc[...] += jnp.sum(oh, axis=0, keepdims=True)
    psum_acc[...] += jnp.sum(probs, axis=0, keepdims=True)

    @pl.when(i == nsteps - 1)
    def _fin():
        t_total = jnp.float32(BLK * nsteps)
        aux_ref[...] = (jnp.sum(cnt_acc[...] * psum_acc[...], keepdims=True)
                        * (NUM_EXPERTS / (t_total * t_total)))


@functools.partial(jax.jit, static_argnames=("interpret",))
def _run(x, W, reputation_scores, expert_loads, expert_counts,
         total_routing_decisions, interpret=False):
    B, S, H = x.shape
    T = B * S
    nsteps = T // BLK
    x2 = x.reshape(T, H)
    wt = W.T
    rep = reputation_scores.reshape(1, NUM_EXPERTS)
    loads = expert_loads.reshape(1, NUM_EXPERTS)
    cnts = expert_counts.reshape(1, NUM_EXPERTS)
    tot = total_routing_decisions.reshape(1, 1)

    w_flat, idx_flat, aux = pl.pallas_call(
        _router_body,
        grid=(nsteps,),
        in_specs=[
            pl.BlockSpec((BLK, H), lambda i: (i, 0)),
            pl.BlockSpec((H, NUM_EXPERTS), lambda i: (0, 0)),
            pl.BlockSpec((1, NUM_EXPERTS), lambda i: (0, 0)),
            pl.BlockSpec((1, NUM_EXPERTS), lambda i: (0, 0)),
            pl.BlockSpec((1, NUM_EXPERTS), lambda i: (0, 0)),
            pl.BlockSpec((1, 1), lambda i: (0, 0)),
        ],
        out_specs=[
            pl.BlockSpec((BLK, TOP_K), lambda i: (i, 0)),
            pl.BlockSpec((BLK, TOP_K), lambda i: (i, 0)),
            pl.BlockSpec((1, 1), lambda i: (0, 0)),
        ],
        out_shape=[
            jax.ShapeDtypeStruct((T, TOP_K), jnp.float32),
            jax.ShapeDtypeStruct((T, TOP_K), jnp.int32),
            jax.ShapeDtypeStruct((1, 1), jnp.float32),
        ],
        scratch_shapes=[
            pltpu.VMEM((1, NUM_EXPERTS), jnp.float32),
            pltpu.VMEM((1, NUM_EXPERTS), jnp.float32),
        ],
        interpret=interpret,
    )(x2, wt, rep, loads, cnts, tot)
    return (w_flat.reshape(B, S, TOP_K),
            idx_flat.reshape(B, S, TOP_K),
            aux[0, 0])


def kernel(x, W, reputation_scores, expert_loads, expert_counts,
           total_routing_decisions):
    return _run(x, W, reputation_scores, expert_loads, expert_counts,
                total_routing_decisions)
